# Initial kernel scaffold; baseline (speedup 1.0000x reference)
import jax
import jax.numpy as jnp
from jax.experimental import pallas as pl


def kernel(x, table):
    raise NotImplementedError("write your pallas kernel here")


# speedup vs baseline: 1.6860x; 1.6860x over previous
import functools

import jax
import jax.numpy as jnp
from jax import lax
from jax.experimental import pallas as pl
from jax.experimental.pallas import tpu as pltpu
from jax.experimental.pallas import tpu_sc as plsc

_CHUNK = 128


@functools.lru_cache(maxsize=None)
def _build(n_chunks: int, d: int):
    info = plsc.get_sparse_core_info()
    nc, ns = info.num_cores, info.num_subcores
    nw = nc * ns

    mesh = plsc.VectorSubcoreMesh(core_axis_name="c", subcore_axis_name="s")
    per_w = n_chunks * _CHUNK

    @functools.partial(
        pl.kernel,
        mesh=mesh,
        compiler_params=pltpu.CompilerParams(use_tc_tiling_on_sc=False),
        out_type=jax.ShapeDtypeStruct((nw * per_w, d), jnp.float32),
        scratch_types=[
            pltpu.VMEM((n_chunks, _CHUNK), jnp.int32),
            pltpu.VMEM((_CHUNK, d), jnp.float32),
            pltpu.SemaphoreType.DMA,
        ],
    )
    def gather_kernel(idx_hbm, table_hbm, out_hbm, idx_v, rows_v, gsem):
        wid = lax.axis_index("s") * nc + lax.axis_index("c")
        base = wid * per_w
        pltpu.sync_copy(idx_hbm.at[wid], idx_v)

        def body(j, carry):
            pltpu.async_copy(table_hbm.at[idx_v.at[j]], rows_v, gsem).wait()
            pltpu.sync_copy(rows_v, out_hbm.at[pl.ds(base + j * _CHUNK, _CHUNK)])
            return carry

        lax.fori_loop(0, n_chunks, body, 0)

    return gather_kernel, nw


def kernel(x, table):
    b, h = x.shape
    v, d = table.shape
    n = b * h

    info = plsc.get_sparse_core_info()
    nw = info.num_cores * info.num_subcores
    tile = nw * _CHUNK
    n_pad = ((n + tile - 1) // tile) * tile

    x_flat = x.reshape(n).astype(jnp.int32)
    if n_pad != n:
        x_flat = jnp.concatenate([x_flat, jnp.zeros(n_pad - n, jnp.int32)])
    n_chunks = n_pad // tile

    gather_kernel, nw = _build(n_chunks, d)
    out = gather_kernel(x_flat.reshape(nw, n_chunks, _CHUNK), table)
    return out[:n].reshape(b, h, d)


# speedup vs baseline: 1.8713x; 1.1099x over previous
import functools

import jax
import jax.numpy as jnp
from jax import lax
from jax.experimental import pallas as pl
from jax.experimental.pallas import tpu as pltpu
from jax.experimental.pallas import tpu_sc as plsc

_CHUNK = 128
_GRP = 4


@functools.lru_cache(maxsize=None)
def _build(n_chunks: int, d: int):
    info = plsc.get_sparse_core_info()
    nc, ns = info.num_cores, info.num_subcores
    nw = nc * ns

    mesh = plsc.VectorSubcoreMesh(core_axis_name="c", subcore_axis_name="s")
    per_w = n_chunks * _CHUNK
    n_groups = n_chunks // _GRP
    grows = _GRP * _CHUNK

    @functools.partial(
        pl.kernel,
        mesh=mesh,
        compiler_params=pltpu.CompilerParams(use_tc_tiling_on_sc=False),
        out_type=jax.ShapeDtypeStruct((nw * per_w, d), jnp.float32),
        scratch_types=[
            pltpu.VMEM((n_chunks, _CHUNK), jnp.int32),
            pltpu.VMEM((2, grows, d), jnp.float32),
            pltpu.SemaphoreType.DMA,
            pltpu.SemaphoreType.DMA,
            pltpu.SemaphoreType.DMA,
            pltpu.SemaphoreType.DMA,
        ],
    )
    def gather_kernel(idx_hbm, table_hbm, out_hbm, idx_v, rows_v, g0, g1, s0, s1):
        gsems = (g0, g1)
        ssems = (s0, s1)
        wid = lax.axis_index("s") * nc + lax.axis_index("c")
        base = wid * per_w
        pltpu.sync_copy(idx_hbm.at[wid], idx_v)

        def gather_desc(g, slot):
            return [
                pltpu.make_async_copy(
                    table_hbm.at[idx_v.at[g * _GRP + c]],
                    rows_v.at[slot].at[pl.ds(c * _CHUNK, _CHUNK)],
                    gsems[slot],
                )
                for c in range(_GRP)
            ]

        def store_desc(g, slot):
            return pltpu.make_async_copy(
                rows_v.at[slot],
                out_hbm.at[pl.ds(base + g * grows, grows)],
                ssems[slot],
            )

        for c in gather_desc(0, 0):
            c.start()

        def body(g2, carry):
            for s in (0, 1):
                g = g2 * 2 + s
                o = 1 - s
                for c in gather_desc(g, s):
                    c.wait()

                @pl.when(g + 1 < n_groups)
                def _():
                    @pl.when(g >= 1)
                    def _():
                        store_desc(g - 1, o).wait()

                    for c in gather_desc(g + 1, o):
                        c.start()

                store_desc(g, s).start()
            return carry

        lax.fori_loop(0, n_groups // 2, body, 0)
        store_desc(n_groups - 2, (n_groups - 2) % 2).wait()
        store_desc(n_groups - 1, (n_groups - 1) % 2).wait()

    return gather_kernel, nw


def kernel(x, table):
    b, h = x.shape
    v, d = table.shape
    n = b * h

    info = plsc.get_sparse_core_info()
    nw = info.num_cores * info.num_subcores
    tile = nw * _CHUNK * _GRP * 2
    n_pad = ((n + tile - 1) // tile) * tile

    x_flat = x.reshape(n).astype(jnp.int32)
    if n_pad != n:
        x_flat = jnp.concatenate([x_flat, jnp.zeros(n_pad - n, jnp.int32)])
    n_chunks = n_pad // (nw * _CHUNK)

    gather_kernel, nw = _build(n_chunks, d)
    out = gather_kernel(x_flat.reshape(nw, n_chunks, _CHUNK), table)
    return out[:n].reshape(b, h, d)
